# baseline (speedup 1.0000x reference)
import functools

import jax
import jax.numpy as jnp
from jax import lax
from jax.experimental import pallas as pl
from jax.experimental.pallas import tpu as pltpu
from jax.experimental.pallas import tpu_sc as plsc

_NC = 2
_NS = 16
_NW = _NC * _NS
_B = 80
_L = 16


@functools.partial(jax.jit, static_argnums=(5, 6, 7))
def _run(eidx2, et, hb, nb, gb, E, epw, nblk):
    D = hb.shape[1]
    G = gb.shape[0]
    nslot = 4
    mesh = plsc.VectorSubcoreMesh(core_axis_name="c", subcore_axis_name="s")

    @functools.partial(
        pl.kernel,
        mesh=mesh,
        out_type=jax.ShapeDtypeStruct((E,), jnp.float32),
        compiler_params=pltpu.CompilerParams(needs_layout_passes=False,
                                             use_tc_tiling_on_sc=False),
        scratch_types=[
            pltpu.VMEM((epw,), jnp.int32),
            pltpu.VMEM((epw,), jnp.int32),
            pltpu.VMEM((epw,), jnp.int32),
            pltpu.VMEM((nslot, _B, D), jnp.bfloat16),
            pltpu.VMEM((G, D), jnp.bfloat16),
            pltpu.VMEM((epw,), jnp.float32),
        ] + [pltpu.SemaphoreType.DMA] * (2 * nslot),
    )
    def k(eidx2_hbm, et_hbm, hb_hbm, nb_hbm, gb_hbm, out_hbm,
          ridx, cidx, eidx, hd_v, g_v, out_v, *sems):
        cid = lax.axis_index("c")
        sid = lax.axis_index("s")
        wid = sid * _NC + cid

        pltpu.sync_copy(eidx2_hbm.at[0, pl.ds(wid * epw, epw)], ridx)
        pltpu.sync_copy(eidx2_hbm.at[1, pl.ds(wid * epw, epw)], cidx)
        pltpu.sync_copy(et_hbm.at[pl.ds(wid * epw, epw)], eidx)
        pltpu.sync_copy(gb_hbm, g_v)

        semA = sems[:nslot]
        semB = sems[nslot:]

        def issue_base(b, j):
            pltpu.async_copy(hb_hbm.at[ridx.at[pl.ds(b * _B, _B)]],
                             hd_v.at[j], semA[j])

        def drain_base(b, j):
            pltpu.make_async_copy(hb_hbm.at[ridx.at[pl.ds(b * _B, _B)]],
                                  hd_v.at[j], semA[j]).wait()

        def issue_add(b, j):
            pltpu.async_copy(nb_hbm.at[cidx.at[pl.ds(b * _B, _B)]],
                             hd_v.at[j], semB[j], add=True)

        def drain_add(b, j):
            pltpu.make_async_copy(nb_hbm.at[cidx.at[pl.ds(b * _B, _B)]],
                                  hd_v.at[j], semB[j]).wait()

        lane = lax.iota(jnp.int32, _L)

        def compute(b, j):
            hd = hd_v.at[j]

            def grp_body(gr, carry):
                base = b * _B + gr * _L
                etv = eidx[pl.ds(base, _L)]
                acc = jnp.zeros((_L,), jnp.float32)
                for kk in range(_L):
                    ge = etv[kk]
                    e = gr * _L + kk
                    s01 = (jnp.abs(hd[e, pl.ds(0, 32)]
                                   + g_v[ge, pl.ds(0, 32)])
                           + jnp.abs(hd[e, pl.ds(32, 32)]
                                     + g_v[ge, pl.ds(32, 32)]))
                    s23 = (jnp.abs(hd[e, pl.ds(64, 32)]
                                   + g_v[ge, pl.ds(64, 32)])
                           + jnp.abs(hd[e, pl.ds(96, 32)]
                                     + g_v[ge, pl.ds(96, 32)]))
                    s = s01 + s23
                    se, so = plsc.unpack(
                        s, format=plsc.PackFormat.INTERLEAVED,
                        preferred_element_type=jnp.float32)
                    acc = jnp.where(lane == kk, jnp.sum(se + so), acc)
                out_v[pl.ds(base, _L)] = acc
                return carry

            lax.fori_loop(0, _B // _L, grp_body, 0)

        for j in range(nslot):
            issue_base(j, j)
        for j in range(nslot - 1):
            drain_base(j, j)
            issue_add(j, j)

        def ring_body(i, carry):
            for j in range(nslot):
                b = i * nslot + j
                drain_add(b, j)
                compute(b, j)

                @pl.when(b + nslot < nblk)
                def _():
                    issue_base(b + nslot, j)

                x = b + nslot - 1
                jp = (j + nslot - 1) % nslot

                @pl.when(x < nblk)
                def _():
                    drain_base(x, jp)
                    issue_add(x, jp)

            return carry

        lax.fori_loop(0, nblk // nslot, ring_body, 0)
        for r in range(nblk % nslot):
            b = (nblk // nslot) * nslot + r
            drain_add(b, b % nslot)
            compute(b, b % nslot)

        pltpu.sync_copy(out_v, out_hbm.at[pl.ds(wid * epw, epw)])

    return k(eidx2, et, hb, nb, gb)


def kernel(h, g, edge_idx, edge_type):
    E = edge_type.shape[0]
    epw = E // _NW
    nblk = epw // _B
    hb = h.astype(jnp.float32).astype(jnp.bfloat16)
    nb = -hb
    gb = g.astype(jnp.float32).astype(jnp.bfloat16)
    return _run(edge_idx.astype(jnp.int32), edge_type.astype(jnp.int32),
                hb, nb, gb, E, epw, nblk)


# speedup vs baseline: 1.1683x; 1.1683x over previous
import functools

import jax
import jax.numpy as jnp
from jax import lax
from jax.experimental import pallas as pl
from jax.experimental.pallas import tpu as pltpu
from jax.experimental.pallas import tpu_sc as plsc

_NC = 2
_NS = 16
_NW = _NC * _NS
_B = 80
_L = 16


@functools.partial(jax.jit, static_argnums=(5, 6, 7))
def _run(eidx2, et, hb, nb, gb, E, epw, nblk):
    D = hb.shape[1]
    G = gb.shape[0]
    nslot = 6
    mesh = plsc.VectorSubcoreMesh(core_axis_name="c", subcore_axis_name="s")

    @functools.partial(
        pl.kernel,
        mesh=mesh,
        out_type=jax.ShapeDtypeStruct((E,), jnp.float32),
        compiler_params=pltpu.CompilerParams(needs_layout_passes=False,
                                             use_tc_tiling_on_sc=False),
        scratch_types=[
            pltpu.VMEM((epw,), jnp.int32),
            pltpu.VMEM((epw,), jnp.int32),
            pltpu.VMEM((epw,), jnp.int32),
            pltpu.VMEM((nslot, _B, D), jnp.bfloat16),
            pltpu.VMEM((G, D), jnp.bfloat16),
            pltpu.VMEM((epw,), jnp.float32),
        ] + [pltpu.SemaphoreType.DMA] * (2 * nslot),
    )
    def k(eidx2_hbm, et_hbm, hb_hbm, nb_hbm, gb_hbm, out_hbm,
          ridx, cidx, eidx, hd_v, g_v, out_v, *sems):
        cid = lax.axis_index("c")
        sid = lax.axis_index("s")
        wid = sid * _NC + cid

        pltpu.sync_copy(eidx2_hbm.at[0, pl.ds(wid * epw, epw)], ridx)
        pltpu.sync_copy(eidx2_hbm.at[1, pl.ds(wid * epw, epw)], cidx)
        pltpu.sync_copy(et_hbm.at[pl.ds(wid * epw, epw)], eidx)
        pltpu.sync_copy(gb_hbm, g_v)

        semA = sems[:nslot]
        semB = sems[nslot:]

        def issue_base(b, j):
            pltpu.async_copy(hb_hbm.at[ridx.at[pl.ds(b * _B, _B)]],
                             hd_v.at[j], semA[j])

        def drain_base(b, j):
            pltpu.make_async_copy(hb_hbm.at[ridx.at[pl.ds(b * _B, _B)]],
                                  hd_v.at[j], semA[j]).wait()

        def issue_add(b, j):
            pltpu.async_copy(nb_hbm.at[cidx.at[pl.ds(b * _B, _B)]],
                             hd_v.at[j], semB[j], add=True)

        def drain_add(b, j):
            pltpu.make_async_copy(nb_hbm.at[cidx.at[pl.ds(b * _B, _B)]],
                                  hd_v.at[j], semB[j]).wait()

        lane = lax.iota(jnp.int32, _L)

        def compute(b, j):
            hd = hd_v.at[j]

            def grp_body(gr, carry):
                base = b * _B + gr * _L
                etv = eidx[pl.ds(base, _L)]
                acc = jnp.zeros((_L,), jnp.float32)
                for kk in range(_L):
                    ge = etv[kk]
                    e = gr * _L + kk
                    s01 = (jnp.abs(hd[e, pl.ds(0, 32)]
                                   + g_v[ge, pl.ds(0, 32)])
                           + jnp.abs(hd[e, pl.ds(32, 32)]
                                     + g_v[ge, pl.ds(32, 32)]))
                    s23 = (jnp.abs(hd[e, pl.ds(64, 32)]
                                   + g_v[ge, pl.ds(64, 32)])
                           + jnp.abs(hd[e, pl.ds(96, 32)]
                                     + g_v[ge, pl.ds(96, 32)]))
                    s = s01 + s23
                    se, so = plsc.unpack(
                        s, format=plsc.PackFormat.INTERLEAVED,
                        preferred_element_type=jnp.float32)
                    acc = jnp.where(lane == kk, jnp.sum(se + so), acc)
                out_v[pl.ds(base, _L)] = acc
                return carry

            lax.fori_loop(0, _B // _L, grp_body, 0)

        for j in range(nslot):
            issue_base(j, j)
        for j in range(2):
            drain_base(j, j)
            issue_add(j, j)

        def ring_body(i, carry):
            for j in range(nslot):
                b = i * nslot + j
                drain_add(b, j)

                x = b + 2
                jp = (j + 2) % nslot

                @pl.when(x < nblk)
                def _():
                    drain_base(x, jp)
                    issue_add(x, jp)

                compute(b, j)

                @pl.when(b + nslot < nblk)
                def _():
                    issue_base(b + nslot, j)

            return carry

        lax.fori_loop(0, nblk // nslot, ring_body, 0)
        for r in range(nblk % nslot):
            b = (nblk // nslot) * nslot + r
            j = b % nslot
            drain_add(b, j)
            if r + 2 < nblk % nslot:
                drain_base(b + 2, (j + 2) % nslot)
                issue_add(b + 2, (j + 2) % nslot)
            compute(b, j)

        pltpu.sync_copy(out_v, out_hbm.at[pl.ds(wid * epw, epw)])

    return k(eidx2, et, hb, nb, gb)


def kernel(h, g, edge_idx, edge_type):
    E = edge_type.shape[0]
    epw = E // _NW
    nblk = epw // _B
    hb = h.astype(jnp.float32).astype(jnp.bfloat16)
    nb = -hb
    gb = g.astype(jnp.float32).astype(jnp.bfloat16)
    return _run(edge_idx.astype(jnp.int32), edge_type.astype(jnp.int32),
                hb, nb, gb, E, epw, nblk)
